# Initial kernel scaffold; baseline (speedup 1.0000x reference)
import jax
import jax.numpy as jnp
from jax.experimental import pallas as pl


def kernel(x, params):
    raise NotImplementedError("write your pallas kernel here")


# speedup vs baseline: 1.0700x; 1.0700x over previous
import math

import jax
import jax.numpy as jnp
from jax.experimental import pallas as pl

D_MODEL = 256
NHEAD = 8
HEAD_DIM = D_MODEL // NHEAD
NLAYERS = 6
DIM_FF = 512
NCH = 142
EMB = 128
MAXLEN = 50
NQ = 4
NCODES = 128

T_PAD = 56
C_PAD = 256
BB = 16
SCALE = 1.0 / math.sqrt(HEAD_DIM)


_ERF_COEF = (0.49416765570640564, -0.3464145064353943, 0.15958236157894135,
             -0.07387414574623108, 0.032677773386240005, -0.013576611876487732,
             0.0052637336775660515, -0.0019008711678907275, 0.000639706791844219,
             -0.00020092404156457633, 5.9057601902168244e-05,
             -1.6257798051810823e-05, 4.232415449223481e-06,
             -1.011215658763831e-06, 2.6297837507627264e-07)
_ERF_SCALE = 0.13149243918474687


def _erf(x):
    ax = jnp.minimum(jnp.abs(x), 3.9)
    u = ax * ax * _ERF_SCALE - 1.0
    b1 = jnp.zeros_like(u)
    b2 = jnp.zeros_like(u)
    for c in _ERF_COEF[:0:-1]:
        b1, b2 = 2.0 * u * b1 - b2 + c, b1
    e = ax * (u * b1 - b2 + _ERF_COEF[0])
    e = jnp.minimum(e, 1.0)
    return jnp.where(x < 0.0, -e, e)


def _exp(x):
    x = jnp.clip(x, -87.0, 88.0)
    n = jnp.round(x * 1.4426950408889634)
    r = x - n * 0.693359375 + n * 2.1219444005469057e-4
    p = 1.0 + r * (1.0 + r * (0.5 + r * (0.16666667 + r * (
        0.041666668 + r * (0.008333334 + r * 0.0013888889)))))
    ni = n.astype(jnp.int32)
    scale = jax.lax.bitcast_convert_type((ni + 127) << 23, jnp.float32)
    return p * scale


def _recip(d):
    r = 1.0 / d
    return r * (2.0 - d * r)


def _ln(x2d, g, b, eps=1e-5):
    m = jnp.mean(x2d, axis=-1, keepdims=True)
    c = x2d - m
    v = jnp.mean(c * c, axis=-1, keepdims=True)
    ve = v + eps
    s = jnp.sqrt(ve)
    s = 0.5 * (s + ve * _recip(s))
    return c * _recip(s) * g + b


def _gelu(x):
    return 0.5 * x * (1.0 + _erf(x * (1.0 / math.sqrt(2.0))))


def _mm(a, b):
    dims = (((1,), (0,)), ((), ()))
    b1 = b.astype(jnp.bfloat16).astype(jnp.float32)
    r = b - b1
    b2 = r.astype(jnp.bfloat16).astype(jnp.float32)
    b3 = (r - b2).astype(jnp.bfloat16).astype(jnp.float32)

    def d1(x, y):
        return jax.lax.dot_general(x, y, dims,
                                   preferred_element_type=jnp.float32)

    return (d1(a, b1) + d1(a, b2)) + d1(a, b3)


def _bmm(a, b, contract):
    return jax.lax.dot_general(a, b, (((2,), (contract,)), ((0,), (0,))),
                               preferred_element_type=jnp.float32)


def _heads(a, rows):
    a = a.reshape(BB, rows, NHEAD, HEAD_DIM)
    a = jnp.swapaxes(a, 1, 2)
    return a.reshape(BB * NHEAD, rows, HEAD_DIM)


def _model_body(x_ref, in_w_ref, posb_ref,
                qkv_w_ref, qkv_b_ref, proj_w_ref, proj_b_ref,
                ln1_g_ref, ln1_b_ref, ln2_g_ref, ln2_b_ref,
                ffn_w1_ref, ffn_b1_ref, ffn_w2_ref, ffn_b2_ref,
                lnf_ref, outw1_ref, outb1_ref, outw2_ref, outb2_ref,
                cb_ref, decw1_ref, decb1_ref, decw2_ref, decb2_ref,
                o_ref):
    R = BB * T_PAD
    x2d = x_ref[...].reshape(R, C_PAD)
    h = _mm(x2d, in_w_ref[...]).reshape(BB, T_PAD, D_MODEL) + posb_ref[...][None]
    h = h.reshape(R, D_MODEL)

    mask = jnp.triu(jnp.full((T_PAD, T_PAD), -1e30, jnp.float32), k=1)

    def layer(l, h):
        hn = _ln(h, ln1_g_ref[l, 0], ln1_b_ref[l, 0])
        qkv = _mm(hn, qkv_w_ref[l]) + qkv_b_ref[l, 0]
        q = _heads(qkv[:, :D_MODEL], T_PAD)
        k = _heads(qkv[:, D_MODEL:2 * D_MODEL], T_PAD)
        v = _heads(qkv[:, 2 * D_MODEL:], T_PAD)
        s = _bmm(q, k, 2) * SCALE + mask[None]
        s = s - jnp.max(s, axis=-1, keepdims=True)
        e = _exp(s)
        p = e * _recip(jnp.sum(e, axis=-1, keepdims=True))
        o = _bmm(p, v, 1)
        o = o.reshape(BB, NHEAD, T_PAD, HEAD_DIM)
        o = jnp.swapaxes(o, 1, 2).reshape(R, D_MODEL)
        h = h + (_mm(o, proj_w_ref[l]) + proj_b_ref[l, 0])
        hn = _ln(h, ln2_g_ref[l, 0], ln2_b_ref[l, 0])
        f = _gelu(_mm(hn, ffn_w1_ref[l]) + ffn_b1_ref[l, 0])
        return h + (_mm(f, ffn_w2_ref[l]) + ffn_b2_ref[l, 0])

    h = jax.lax.fori_loop(0, NLAYERS - 1, layer, h, unroll=False)

    L = NLAYERS - 1
    hn = _ln(h, ln1_g_ref[L, 0], ln1_b_ref[L, 0])
    kv = _mm(hn, qkv_w_ref[L][:, D_MODEL:]) + qkv_b_ref[L, 0][D_MODEL:]
    hn_l = hn.reshape(BB, T_PAD, D_MODEL)[:, MAXLEN - 1]
    q_l = _mm(hn_l, qkv_w_ref[L][:, :D_MODEL]) + qkv_b_ref[L, 0][:D_MODEL]
    k = _heads(kv[:, :D_MODEL], T_PAD)
    v = _heads(kv[:, D_MODEL:], T_PAD)
    qb = q_l.reshape(BB, 1, NHEAD, HEAD_DIM)
    qb = jnp.swapaxes(qb, 1, 2).reshape(BB * NHEAD, 1, HEAD_DIM)
    lmask = jnp.where(
        jax.lax.broadcasted_iota(jnp.int32, (1, T_PAD), 1) > MAXLEN - 1,
        -1e30, 0.0).astype(jnp.float32)
    s = _bmm(qb, k, 2) * SCALE + lmask[None]
    s = s - jnp.max(s, axis=-1, keepdims=True)
    e = _exp(s)
    p = e * _recip(jnp.sum(e, axis=-1, keepdims=True))
    o = _bmm(p, v, 1)
    o = o.reshape(BB, NHEAD, 1, HEAD_DIM)
    o = jnp.swapaxes(o, 1, 2).reshape(BB, D_MODEL)
    h_l = h.reshape(BB, T_PAD, D_MODEL)[:, MAXLEN - 1]
    h_l = h_l + (_mm(o, proj_w_ref[L]) + proj_b_ref[L, 0])
    hn2 = _ln(h_l, ln2_g_ref[L, 0], ln2_b_ref[L, 0])
    f = _gelu(_mm(hn2, ffn_w1_ref[L]) + ffn_b1_ref[L, 0])
    h_l = h_l + (_mm(f, ffn_w2_ref[L]) + ffn_b2_ref[L, 0])

    hl = _ln(h_l, lnf_ref[0], lnf_ref[1])
    z = _mm(_gelu(_mm(hl, outw1_ref[...]) + outb1_ref[0]),
            outw2_ref[...]) + outb2_ref[0]

    resid = z
    zq_total = jnp.zeros_like(z)
    codes = jnp.arange(NCODES, dtype=jnp.int32)
    for i in range(NQ):
        cb = cb_ref[i]
        d = (jnp.sum(resid * resid, axis=1, keepdims=True)
             + jnp.sum(cb * cb, axis=1)[None, :]
             - 2.0 * jax.lax.dot_general(
                 resid, cb, (((1,), (1,)), ((), ())),
                 preferred_element_type=jnp.float32))
        idx = jnp.argmin(d, axis=1)
        onehot = (codes[None, :] == idx[:, None]).astype(jnp.float32)
        zq = _mm(onehot, cb)
        zq_st = resid + (zq - resid)
        zq_total = zq_total + zq_st
        resid = resid - zq_st

    pred = _mm(_gelu(_mm(zq_total, decw1_ref[...]) + decb1_ref[0]),
               decw2_ref[...]) + decb2_ref[0]
    o_ref[...] = pred


@jax.jit
def kernel(x, params):
    p = params
    B = x.shape[0]
    nblk = B // BB

    xp = jnp.pad(x, ((0, 0), (0, T_PAD - MAXLEN), (0, C_PAD - NCH)))
    in_w = jnp.pad(p['in_W'], ((0, C_PAD - NCH), (0, 0)))
    posb = (jnp.pad(p['pos'][0], ((0, T_PAD - MAXLEN), (0, 0)))
            + p['in_b'][None, :])

    blocks = p['blocks']
    stk = lambda n: jnp.stack([b[n] for b in blocks])
    stk1 = lambda n: jnp.stack([b[n][None, :] for b in blocks])
    lnf = jnp.stack([p['lnf_g'], p['lnf_b']])

    w_args = (in_w, posb,
              stk('qkv_W'), stk1('qkv_b'), stk('proj_W'), stk1('proj_b'),
              stk1('ln1_g'), stk1('ln1_b'), stk1('ln2_g'), stk1('ln2_b'),
              stk('ffn_W1'), stk1('ffn_b1'), stk('ffn_W2'), stk1('ffn_b2'),
              lnf, p['out_W1'], p['out_b1'][None, :],
              p['out_W2'], p['out_b2'][None, :], p['codebooks'],
              p['dec_W1'], p['dec_b1'][None, :],
              jnp.pad(p['dec_W2'], ((0, 0), (0, C_PAD - NCH))),
              jnp.pad(p['dec_b2'], (0, C_PAD - NCH))[None, :])

    def const_spec(a):
        nd = a.ndim
        return pl.BlockSpec(a.shape, lambda *_: (0,) * nd)

    out = pl.pallas_call(
        _model_body,
        out_shape=jax.ShapeDtypeStruct((B, C_PAD), jnp.float32),
        grid=(nblk,),
        in_specs=[pl.BlockSpec((BB, T_PAD, C_PAD), lambda i: (i, 0, 0))]
                 + [const_spec(a) for a in w_args],
        out_specs=pl.BlockSpec((BB, C_PAD), lambda i: (i, 0)),
    )(xp, *w_args)
    return out[:, :NCH]
